# baseline (speedup 1.0000x reference)
import jax
import jax.numpy as jnp
from jax import lax
from jax.experimental import pallas as pl
from jax.experimental.pallas import tpu as pltpu
from jax.experimental.pallas import tpu_sc as plsc

B, S, H = 4, 2048, 768
NC, NS, L = 2, 16, 16
NW = NC * NS
PB = S // NW
CH = 64
NCHUNK = (B * PB) // CH
NP = H // (2 * L)
HW = H // 2
EPS = 1e-12


def _lane_shuffle(v, perm):
    dnums = lax.GatherDimensionNumbers(
        offset_dims=(), collapsed_slice_dims=(0,), start_index_map=(0,))
    return lax.gather(v, perm[:, None], dnums, (1,),
                      mode=lax.GatherScatterMode.PROMISE_IN_BOUNDS)


def _rsqrt(var):
    iv = plsc.bitcast(var, jnp.int32)
    y = plsc.bitcast(jnp.int32(0x5F3759DF) - (iv >> 1), jnp.float32)
    for _ in range(2):
        y = y * (1.5 - 0.5 * var * y * y)
    return y


def _interleave_bf16_words(x):
    n = x.shape[0]
    ilv = (x.astype(jnp.bfloat16)
           .reshape(n, NP, 2, L)
           .swapaxes(2, 3)
           .reshape(n, H // 2, 2))
    return lax.bitcast_convert_type(ilv, jnp.int32).reshape(-1)


def _body(ids_h, tt_h, word_h, pos_h, tok_h, out_h,
          posbuf, tokbuf, wb0, wb1, idsall, ttall,
          g0, g1, o0, o1):
    cid = lax.axis_index("c")
    sid = lax.axis_index("s")
    wid = sid * NC + cid
    p0 = wid * PB

    wbufs = [wb0, wb1]
    gsems = [g0, g1]
    osems = [o0, o1]
    gcopies = [None, None]
    ocopies = [None, None]

    ilv = plsc.PackFormat.INTERLEAVED

    for b in range(B):
        pltpu.sync_copy(ids_h.at[b, pl.ds(p0, PB)], idsall.at[b])
        pltpu.sync_copy(tt_h.at[b, pl.ds(p0, PB)], ttall.at[b, pl.ds(0, PB)])
    pltpu.sync_copy(pos_h.at[pl.ds(p0 * HW, PB * HW)], posbuf)
    pltpu.sync_copy(tok_h, tokbuf)

    def issue_gather(k):
        p = k % 2
        gcopies[p] = pltpu.async_copy(
            word_h.at[idsall.at[k]], wbufs[p], gsems[p])

    issue_gather(0)

    for k in range(NCHUNK):
        p = k % 2
        base = k * S + p0
        if k + 1 < NCHUNK:
            if ocopies[1 - p] is not None:
                ocopies[1 - p].wait()
            issue_gather(k + 1)
        gcopies[p].wait()
        buf = wbufs[p]

        @plsc.parallel_loop(0, CH, 1, unroll=8)
        def token_body(j):
            t = ttall[k, pl.ds(j, L)][0]
            pbase = pl.multiple_of(j * HW, L)
            tbase = pl.multiple_of(t * HW, L)
            acc_a = jnp.zeros((L,), jnp.float32)
            acc_b = jnp.zeros((L,), jnp.float32)
            sq_a = jnp.zeros((L,), jnp.float32)
            sq_b = jnp.zeros((L,), jnp.float32)
            for c in range(NP):
                wa = buf[j, pl.ds(2 * c * L, L)]
                wb = buf[j, pl.ds((2 * c + 1) * L, L)]
                wp = plsc.pack(wa, wb, format=ilv)
                pv = plsc.bitcast(posbuf[pl.ds(pbase + c * L, L)], jnp.bfloat16)
                tv = plsc.bitcast(tokbuf[pl.ds(tbase + c * L, L)], jnp.bfloat16)
                v = (wp + pv) + tv
                buf[j, pl.ds(c * L, L)] = plsc.bitcast(v, jnp.float32)
                va, vb = plsc.unpack(v, format=ilv)
                acc_a = acc_a + va
                acc_b = acc_b + vb
                sq_a = sq_a + va * va
                sq_b = sq_b + vb * vb
            s1 = acc_a + acc_b
            s2 = sq_a + sq_b
            lanes = lax.iota(jnp.int32, L)
            for step in (1, 2, 4, 8):
                perm = lanes ^ step
                s1 = s1 + _lane_shuffle(s1, perm)
                s2 = s2 + _lane_shuffle(s2, perm)
            mf = s1 * (1.0 / H)
            varf = s2 * (1.0 / H) - mf * mf + EPS
            r16 = _rsqrt(varf)
            mean = plsc.pack(mf, mf, format=ilv)
            r = plsc.pack(r16, r16, format=ilv)
            for c in reversed(range(NP)):
                xv = plsc.bitcast(buf[j, pl.ds(c * L, L)], jnp.bfloat16)
                yv = (xv - mean) * r
                ya, yb = plsc.unpack(yv, format=ilv)
                buf[j, pl.ds(2 * c * L, L)] = ya
                buf[j, pl.ds((2 * c + 1) * L, L)] = yb

        ocopies[p] = pltpu.async_copy(buf, out_h.at[pl.ds(base, CH)], osems[p])

    for p in range(2):
        ocopies[p].wait()


def kernel(input_ids, token_type_ids, word_emb, pos_emb, tok_emb, ln_weight, ln_bias):
    del ln_weight, ln_bias
    ids2 = input_ids.astype(jnp.int32)
    tt2 = token_type_ids.astype(jnp.int32)
    pos_i = _interleave_bf16_words(pos_emb)
    tok_i = _interleave_bf16_words(tok_emb)
    mesh = plsc.VectorSubcoreMesh(core_axis_name="c", subcore_axis_name="s")
    out = pl.kernel(
        _body,
        out_type=jax.ShapeDtypeStruct((B * S, H), jnp.float32),
        mesh=mesh,
        compiler_params=pltpu.CompilerParams(needs_layout_passes=False),
        scratch_types=[
            pltpu.VMEM((PB * HW,), jnp.int32),
            pltpu.VMEM((2 * HW,), jnp.int32),
            pltpu.VMEM((CH, H), jnp.float32),
            pltpu.VMEM((CH, H), jnp.float32),
            pltpu.VMEM((B, PB), jnp.int32),
            pltpu.VMEM((B, PB + L), jnp.int32),
            pltpu.SemaphoreType.DMA,
            pltpu.SemaphoreType.DMA,
            pltpu.SemaphoreType.DMA,
            pltpu.SemaphoreType.DMA,
        ],
    )(ids2, tt2, word_emb, pos_i, tok_i)
    return out.reshape(B, S, H)
